# baseline (speedup 1.0000x reference)
import functools

import jax
import jax.numpy as jnp
from jax import lax
from jax.experimental import pallas as pl
from jax.experimental.pallas import tpu as pltpu, tpu_sc as plsc

_RADIUS = 0.2
_NSAMPLE = 32
_B = 4
_N = 4096
_NW = 32
_ROWS_PER_W = (_B * _N) // _NW
_W_PER_B = _N // _ROWS_PER_W
_L = 16
_R2 = _RADIUS * _RADIUS


def _sc_ball_query():
    mesh = plsc.VectorSubcoreMesh(core_axis_name="c", subcore_axis_name="s")

    @functools.partial(
        pl.kernel,
        out_type=jax.ShapeDtypeStruct((_B * _N * _NSAMPLE,), jnp.int32),
        mesh=mesh,
        scratch_types=[
            pltpu.VMEM((_N,), jnp.float32),
            pltpu.VMEM((_N,), jnp.float32),
            pltpu.VMEM((_N,), jnp.float32),
            pltpu.VMEM((320,), jnp.int32),
            pltpu.VMEM((_ROWS_PER_W * _NSAMPLE,), jnp.int32),
        ],
        compiler_params=pltpu.CompilerParams(needs_layout_passes=False),
    )
    def k(xyz_hbm, out_hbm, cx, cy, cz, row, outbuf):
        wid = lax.axis_index("c") * 16 + lax.axis_index("s")
        b = wid // _W_PER_B
        r0 = (wid % _W_PER_B) * _ROWS_PER_W

        pltpu.sync_copy(xyz_hbm.at[pl.ds((b * 3 + 0) * _N, _N)], cx)
        pltpu.sync_copy(xyz_hbm.at[pl.ds((b * 3 + 1) * _N, _N)], cy)
        pltpu.sync_copy(xyz_hbm.at[pl.ds((b * 3 + 2) * _N, _N)], cz)

        iota = jnp.arange(_L, dtype=jnp.int32)
        zeros = jnp.zeros((_L,), jnp.int32)

        def per_query(qi, carry):
            q = r0 + qi
            qsplat = jnp.full((_L,), q, jnp.int32)
            qx = plsc.load_gather(cx, [qsplat])
            qy = plsc.load_gather(cy, [qsplat])
            qz = plsc.load_gather(cz, [qsplat])

            def cond(state):
                j, cnt = state
                return jnp.logical_and(j < _N, cnt < _NSAMPLE)

            def body(state):
                j, cnt = state
                for g in range(4):
                    ms, pcs = [], []
                    for u in range(4):
                        sl = pl.ds(j + (g * 4 + u) * _L, _L)
                        dx = cx[sl] - qx
                        dy = cy[sl] - qy
                        dz = cz[sl] - qz
                        d2 = (dx * dx + dy * dy) + dz * dz
                        m = d2 < _R2
                        ms.append(m)
                        pc = plsc.all_reduce_population_count(m)
                        pcs.append(lax.squeeze(lax.slice(pc, (0,), (1,)), (0,)))
                    for u in range(4):
                        idxv = iota + (j + (g * 4 + u) * _L)
                        plsc.store_compressed(row.at[pl.ds(cnt, _L)], idxv,
                                              mask=ms[u])
                        cnt = cnt + pcs[u]
                return j + 16 * _L, cnt

            _, cnt = lax.while_loop(cond, body, (jnp.int32(0), jnp.int32(0)))

            r0v = row[pl.ds(0, _L)]
            r1v = row[pl.ds(_L, _L)]
            first = jnp.min(jnp.where(iota < cnt, r0v, jnp.int32(1 << 30)))
            o0 = jnp.where(iota < cnt, r0v, first)
            o1 = jnp.where(iota + _L < cnt, r1v, first)
            obase = qi * _NSAMPLE
            outbuf[pl.ds(obase, _L)] = o0
            outbuf[pl.ds(obase + _L, _L)] = o1
            return carry

        lax.fori_loop(0, _ROWS_PER_W, per_query, 0)
        out_off = (b * _N + r0) * _NSAMPLE
        pltpu.sync_copy(outbuf, out_hbm.at[pl.ds(out_off, _ROWS_PER_W * _NSAMPLE)])

    return k


def kernel(xyz, xyz_new):
    del xyz_new
    xyz_flat = jnp.transpose(xyz, (0, 2, 1)).reshape(-1)
    out_flat = _sc_ball_query()(xyz_flat)
    return out_flat.reshape(_B, _N, _NSAMPLE)


# speedup vs baseline: 1.6982x; 1.6982x over previous
import functools

import jax
import jax.numpy as jnp
from jax import lax
from jax.experimental import pallas as pl
from jax.experimental.pallas import tpu as pltpu, tpu_sc as plsc

_RADIUS = 0.2
_NSAMPLE = 32
_B = 4
_N = 4096
_NW = 32
_ROWS_PER_W = (_B * _N) // _NW
_W_PER_B = _N // _ROWS_PER_W
_L = 16
_R2 = _RADIUS * _RADIUS


def _sc_ball_query():
    mesh = plsc.VectorSubcoreMesh(core_axis_name="c", subcore_axis_name="s")

    @functools.partial(
        pl.kernel,
        out_type=jax.ShapeDtypeStruct((_B * _N * _NSAMPLE,), jnp.int32),
        mesh=mesh,
        scratch_types=[
            pltpu.VMEM((_N,), jnp.float32),
            pltpu.VMEM((_N,), jnp.float32),
            pltpu.VMEM((_N,), jnp.float32),
            pltpu.VMEM((320,), jnp.int32),
            pltpu.VMEM((_ROWS_PER_W * _NSAMPLE,), jnp.int32),
        ],
        compiler_params=pltpu.CompilerParams(needs_layout_passes=False),
    )
    def k(xyz_hbm, out_hbm, cx, cy, cz, row, outbuf):
        wid = lax.axis_index("c") * 16 + lax.axis_index("s")
        b = wid // _W_PER_B
        r0 = (wid % _W_PER_B) * _ROWS_PER_W

        pltpu.sync_copy(xyz_hbm.at[pl.ds((b * 3 + 0) * _N, _N)], cx)
        pltpu.sync_copy(xyz_hbm.at[pl.ds((b * 3 + 1) * _N, _N)], cy)
        pltpu.sync_copy(xyz_hbm.at[pl.ds((b * 3 + 2) * _N, _N)], cz)

        iota = jnp.arange(_L, dtype=jnp.int32)
        zeros = jnp.zeros((_L,), jnp.int32)

        def per_query(qi, carry):
            q = r0 + qi
            qsplat = jnp.full((_L,), q, jnp.int32)
            qx = plsc.load_gather(cx, [qsplat])
            qy = plsc.load_gather(cy, [qsplat])
            qz = plsc.load_gather(cz, [qsplat])

            def cond(state):
                j, cnt = state
                return jnp.logical_and(j < _N, cnt < _NSAMPLE)

            def body(state):
                j, cnt = state
                ms, pcs = [], []
                for u in range(16):
                    sl = pl.ds(j + u * _L, _L)
                    dx = cx[sl] - qx
                    dy = cy[sl] - qy
                    dz = cz[sl] - qz
                    d2 = (dx * dx + dy * dy) + dz * dz
                    m = d2 < _R2
                    ms.append(m)
                    pc = plsc.all_reduce_population_count(m)
                    pcs.append(lax.squeeze(lax.slice(pc, (0,), (1,)), (0,)))
                for u in range(16):
                    idxv = iota + (j + u * _L)
                    plsc.store_compressed(row.at[pl.ds(cnt, _L)], idxv,
                                          mask=ms[u])
                    cnt = cnt + pcs[u]
                return j + 16 * _L, cnt

            _, cnt = lax.while_loop(cond, body, (jnp.int32(0), jnp.int32(0)))

            r0v = row[pl.ds(0, _L)]
            r1v = row[pl.ds(_L, _L)]
            first = jnp.min(jnp.where(iota < cnt, r0v, jnp.int32(1 << 30)))
            o0 = jnp.where(iota < cnt, r0v, first)
            o1 = jnp.where(iota + _L < cnt, r1v, first)
            obase = qi * _NSAMPLE
            outbuf[pl.ds(obase, _L)] = o0
            outbuf[pl.ds(obase + _L, _L)] = o1
            return carry

        lax.fori_loop(0, _ROWS_PER_W, per_query, 0)
        out_off = (b * _N + r0) * _NSAMPLE
        pltpu.sync_copy(outbuf, out_hbm.at[pl.ds(out_off, _ROWS_PER_W * _NSAMPLE)])

    return k


def kernel(xyz, xyz_new):
    del xyz_new
    xyz_flat = jnp.transpose(xyz, (0, 2, 1)).reshape(-1)
    out_flat = _sc_ball_query()(xyz_flat)
    return out_flat.reshape(_B, _N, _NSAMPLE)


# speedup vs baseline: 1.7199x; 1.0128x over previous
import functools

import jax
import jax.numpy as jnp
from jax import lax
from jax.experimental import pallas as pl
from jax.experimental.pallas import tpu as pltpu, tpu_sc as plsc

_RADIUS = 0.2
_NSAMPLE = 32
_B = 4
_N = 4096
_NW = 32
_ROWS_PER_W = (_B * _N) // _NW
_W_PER_B = _N // _ROWS_PER_W
_L = 16
_R2 = _RADIUS * _RADIUS


def _sc_ball_query():
    mesh = plsc.VectorSubcoreMesh(core_axis_name="c", subcore_axis_name="s")

    @functools.partial(
        pl.kernel,
        out_type=jax.ShapeDtypeStruct((_B * _N * _NSAMPLE,), jnp.int32),
        mesh=mesh,
        scratch_types=[
            pltpu.VMEM((_N,), jnp.float32),
            pltpu.VMEM((_N,), jnp.float32),
            pltpu.VMEM((_N,), jnp.float32),
            pltpu.VMEM((1024,), jnp.int32),
            pltpu.VMEM((_ROWS_PER_W * _NSAMPLE,), jnp.int32),
        ],
        compiler_params=pltpu.CompilerParams(needs_layout_passes=False),
    )
    def k(xyz_hbm, out_hbm, cx, cy, cz, row, outbuf):
        wid = lax.axis_index("c") * 16 + lax.axis_index("s")
        b = wid // _W_PER_B
        r0 = (wid % _W_PER_B) * _ROWS_PER_W

        pltpu.sync_copy(xyz_hbm.at[pl.ds((b * 3 + 0) * _N, _N)], cx)
        pltpu.sync_copy(xyz_hbm.at[pl.ds((b * 3 + 1) * _N, _N)], cy)
        pltpu.sync_copy(xyz_hbm.at[pl.ds((b * 3 + 2) * _N, _N)], cz)

        iota = jnp.arange(_L, dtype=jnp.int32)
        zeros = jnp.zeros((_L,), jnp.int32)

        def per_query(qi, carry):
            q = r0 + qi
            qsplat = jnp.full((_L,), q, jnp.int32)
            qx = plsc.load_gather(cx, [qsplat])
            qy = plsc.load_gather(cy, [qsplat])
            qz = plsc.load_gather(cz, [qsplat])

            def scan_block(jbase, cnt):
                ms, pcs = [], []
                for u in range(16):
                    sl = pl.ds(jbase + u * _L, _L)
                    dx = cx[sl] - qx
                    dy = cy[sl] - qy
                    dz = cz[sl] - qz
                    d2 = (dx * dx + dy * dy) + dz * dz
                    m = d2 < _R2
                    ms.append(m)
                    pc = plsc.all_reduce_population_count(m)
                    pcs.append(lax.squeeze(lax.slice(pc, (0,), (1,)), (0,)))
                for u in range(16):
                    idxv = iota + (jbase + u * _L)
                    plsc.store_compressed(row.at[pl.ds(cnt, _L)], idxv,
                                          mask=ms[u])
                    cnt = cnt + pcs[u]
                return cnt

            cnt = jnp.int32(0)
            for t in range(3):
                cnt = scan_block(t * 16 * _L, cnt)

            def cond(state):
                j, cnt = state
                return jnp.logical_and(j < _N, cnt < _NSAMPLE)

            def body(state):
                j, cnt = state
                return j + 16 * _L, scan_block(j, cnt)

            _, cnt = lax.while_loop(cond, body, (jnp.int32(48 * _L), cnt))

            r0v = row[pl.ds(0, _L)]
            r1v = row[pl.ds(_L, _L)]
            first = jnp.min(jnp.where(iota < cnt, r0v, jnp.int32(1 << 30)))
            o0 = jnp.where(iota < cnt, r0v, first)
            o1 = jnp.where(iota + _L < cnt, r1v, first)
            obase = qi * _NSAMPLE
            outbuf[pl.ds(obase, _L)] = o0
            outbuf[pl.ds(obase + _L, _L)] = o1
            return carry

        lax.fori_loop(0, _ROWS_PER_W, per_query, 0)
        out_off = (b * _N + r0) * _NSAMPLE
        pltpu.sync_copy(outbuf, out_hbm.at[pl.ds(out_off, _ROWS_PER_W * _NSAMPLE)])

    return k


def kernel(xyz, xyz_new):
    del xyz_new
    xyz_flat = jnp.transpose(xyz, (0, 2, 1)).reshape(-1)
    out_flat = _sc_ball_query()(xyz_flat)
    return out_flat.reshape(_B, _N, _NSAMPLE)
